# Initial kernel scaffold; baseline (speedup 1.0000x reference)
import jax
import jax.numpy as jnp
from jax.experimental import pallas as pl


def kernel(features, W_self1, W_neigh1, W_self2, W_neigh2, adj, batch1):
    raise NotImplementedError("write your pallas kernel here")


# speedup vs baseline: 1.9091x; 1.9091x over previous
import functools

import jax
import jax.numpy as jnp
from jax import lax
from jax.experimental import pallas as pl
from jax.experimental.pallas import tpu as pltpu
from jax.experimental.pallas import tpu_sc as plsc

_N = 100000
_D = 128
_MAX_DEG = 32
_B = 512
_NS1 = 25
_NS2 = 10
_DIM = 128

_NC = 2
_NSC = 16
_NW = _NC * _NSC
_BPW = _B // _NW
_S1W = _BPW * _NS2
_PCH = 16
_RCH = _PCH * _NS1
_NCH = _S1W // _PCH
_IDXW = 80
_LANES = 16


def _sc_sample_gather_agg(features, adj4, batch1):
    mesh = plsc.VectorSubcoreMesh(core_axis_name="c", subcore_axis_name="s",
                                  num_cores=_NC, num_subcores=_NSC)

    @functools.partial(
        pl.kernel,
        out_type=(
            jax.ShapeDtypeStruct((_B, _D), jnp.float32),
            jax.ShapeDtypeStruct((_B * _NS2, _D), jnp.float32),
            jax.ShapeDtypeStruct((_B * _NS2, _D), jnp.float32),
        ),
        mesh=mesh,
        compiler_params=pltpu.CompilerParams(needs_layout_passes=False),
        scratch_types=[
            pltpu.VMEM((_BPW,), jnp.int32),
            pltpu.VMEM((_BPW,), jnp.int32),
            pltpu.VMEM((_BPW,), jnp.int32),
            pltpu.VMEM((_BPW, 4 * _MAX_DEG), jnp.int32),
            pltpu.VMEM((2, _IDXW), jnp.int32),
            pltpu.VMEM((2, _IDXW), jnp.int32),
            pltpu.VMEM((_S1W,), jnp.int32),
            pltpu.VMEM((_S1W, 4 * _MAX_DEG), jnp.int32),
            pltpu.VMEM((5, _IDXW), jnp.int32),
            pltpu.VMEM((_BPW, _D), jnp.float32),
            pltpu.VMEM((_S1W, _D), jnp.float32),
            pltpu.VMEM((_RCH, _D), jnp.float32),
            pltpu.VMEM((_S1W, _D), jnp.float32),
            pltpu.SemaphoreType.DMA,
        ],
    )
    def k(feat_hbm, adj_hbm, batch_hbm, h0_out, h1_out, h2s_out,
          bq_v, boff_v, bidx_v, adj0_v, s1_v, s1q_v, off1_v, adj1_v, s2_v,
          h0_v, h1_v, gbuf, h2s_v, sem):
        wid = lax.axis_index("s") * _NC + lax.axis_index("c")
        base_b = wid * _BPW
        base_s1 = wid * _S1W
        lane = lax.iota(jnp.int32, _LANES)

        pltpu.sync_copy(batch_hbm.at[pl.ds(base_b, _BPW)], bidx_v)
        b = bidx_v[...]
        bq_v[...] = lax.shift_right_logical(b, 2)
        boff_v[...] = lax.shift_left(lax.bitwise_and(b, 3), 5)
        cp_adj0 = pltpu.async_copy(adj_hbm.at[bq_v], adj0_v, sem)
        cp_h0 = pltpu.async_copy(feat_hbm.at[bidx_v], h0_v, sem)
        cp_adj0.wait()
        cp_h0.wait()

        for g in range(_S1W // _LANES):
            j = g * _LANES + lane
            r = lax.div(j, _NS2)
            c = lax.rem(j, _NS2)
            col = plsc.load_gather(boff_v, [r]) + c
            vals = plsc.load_gather(adj0_v, [r, col])
            s1_v[g // 5, pl.ds((g % 5) * _LANES, _LANES)] = vals
            s1q_v[g // 5, pl.ds((g % 5) * _LANES, _LANES)] = (
                lax.shift_right_logical(vals, 2))
            off1_v[pl.ds(g * _LANES, _LANES)] = (
                lax.shift_left(lax.bitwise_and(vals, 3), 5))

        cps = []
        for t in range(2):
            dst = pl.ds(t * _IDXW, _IDXW)
            cps.append(pltpu.async_copy(feat_hbm.at[s1_v.at[t]], h1_v.at[dst], sem))
            cps.append(pltpu.async_copy(adj_hbm.at[s1q_v.at[t]], adj1_v.at[dst], sem))
        for cp in cps:
            cp.wait()

        pltpu.sync_copy(h0_v, h0_out.at[pl.ds(base_b, _BPW)])
        pltpu.sync_copy(h1_v, h1_out.at[pl.ds(base_s1, _S1W)])

        def chunk_body(cc, carry):
            def idx_body(g, c2):
                j = g * _LANES + lane
                p = lax.div(j, _NS1)
                c = lax.rem(j, _NS1)
                pg = cc * _PCH + p
                col = plsc.load_gather(off1_v, [pg]) + c
                vals = plsc.load_gather(adj1_v, [pg, col])
                s2_v[lax.div(g, 5), pl.ds(lax.rem(g, 5) * _LANES, _LANES)] = vals
                return c2
            lax.fori_loop(0, _RCH // _LANES, idx_body, 0)

            cps2 = [
                pltpu.async_copy(feat_hbm.at[s2_v.at[t]],
                                 gbuf.at[pl.ds(t * _IDXW, _IDXW)], sem)
                for t in range(_RCH // _IDXW)
            ]
            for cp in cps2:
                cp.wait()

            def parent_body(p, c2):
                row0 = p * _NS1
                def r_body(r, accs):
                    row = row0 + r
                    return tuple(
                        accs[d] + gbuf[row, pl.ds(d * _LANES, _LANES)]
                        for d in range(_D // _LANES)
                    )
                accs = lax.fori_loop(
                    0, _NS1, r_body,
                    tuple(jnp.zeros((_LANES,), jnp.float32)
                          for _ in range(_D // _LANES)))
                for d in range(_D // _LANES):
                    h2s_v[cc * _PCH + p, pl.ds(d * _LANES, _LANES)] = accs[d]
                return c2
            lax.fori_loop(0, _PCH, parent_body, 0)
            return carry

        lax.fori_loop(0, _NCH, chunk_body, 0)
        pltpu.sync_copy(h2s_v, h2s_out.at[pl.ds(base_s1, _S1W)])

    return k(features, adj4, batch1)


def _tc_dense(h0, h1, h2s, W_self1, W_neigh1, W_self2, W_neigh2):
    def body(h0_r, h1_r, h2s_r, ws1_r, wn1_r, ws2_r, wn2_r, out_r):
        f32 = jnp.float32
        h1v = h1_r[...]
        nh1 = jnp.maximum(jnp.concatenate([
            jnp.dot(h1v, ws1_r[...], preferred_element_type=f32),
            jnp.dot(h2s_r[...] * (1.0 / _NS1), wn1_r[...],
                    preferred_element_type=f32)], axis=1), 0.0)
        nh1m = jnp.mean(nh1.reshape(_B, _NS2, 2 * _DIM), axis=1)
        h1m = jnp.mean(h1v.reshape(_B, _NS2, _D), axis=1)
        nh0 = jnp.maximum(jnp.concatenate([
            jnp.dot(h0_r[...], ws1_r[...], preferred_element_type=f32),
            jnp.dot(h1m, wn1_r[...], preferred_element_type=f32)],
            axis=1), 0.0)
        o = jnp.concatenate([
            jnp.dot(nh0, ws2_r[...], preferred_element_type=f32),
            jnp.dot(nh1m, wn2_r[...], preferred_element_type=f32)], axis=1)
        nrm = jnp.sqrt(jnp.sum(o * o, axis=1, keepdims=True))
        out_r[...] = o / jnp.maximum(nrm, 1e-12)

    return pl.pallas_call(
        body,
        out_shape=jax.ShapeDtypeStruct((_B, 2 * _DIM), jnp.float32),
    )(h0, h1, h2s, W_self1, W_neigh1, W_self2, W_neigh2)


@jax.jit
def kernel(features, W_self1, W_neigh1, W_self2, W_neigh2, adj, batch1):
    batch1 = batch1.astype(jnp.int32)
    adj4 = adj.reshape(_N // 4, 4 * _MAX_DEG)
    h0, h1, h2s = _sc_sample_gather_agg(features, adj4, batch1)
    return _tc_dense(h0, h1, h2s, W_self1, W_neigh1, W_self2, W_neigh2)
